# baseline (speedup 1.0000x reference)
import functools

import jax
import jax.numpy as jnp
from jax import lax
from jax.experimental import pallas as pl
from jax.experimental.pallas import tpu as pltpu
from jax.experimental.pallas import tpu_sc as plsc

HEADS = 4
KNN = 16
NK = 512
HALF = 256
KD = 512
BS = 4096
VD = 512
HK = HEADS * KNN
BLK = 128
NEG = -1e30


def _mono_i32(x):
    b = lax.bitcast_convert_type(x, jnp.int32)
    return b ^ (lax.shift_right_arithmetic(b, 31) & jnp.int32(0x7FFFFFFF))


def _inv_mono(k):
    b = k ^ (lax.shift_right_arithmetic(k, 31) & jnp.int32(0x7FFFFFFF))
    return lax.bitcast_convert_type(b, jnp.float32)


def _topk16(s, payload=None):
    n = s.shape[1]
    iota = lax.broadcasted_iota(jnp.int32, s.shape, 1)
    pay = iota if payload is None else payload
    key = _mono_i32(s)
    ms, fps = [], []
    for _ in range(KNN):
        m = jnp.max(key, axis=1, keepdims=True)
        eq = key == m
        fps.append(jnp.min(jnp.where(eq, pay, jnp.int32(2**31 - 1)),
                           axis=1, keepdims=True))
        ms.append(m)
        key = jnp.where(eq, jnp.int32(-2**31), key)
    vals = _inv_mono(jnp.concatenate(ms, axis=1))
    return vals, jnp.concatenate(fps, axis=1)


def _tc_body(x_ref, wqt_ref, bq_ref, keys_ref, idx_ref, w_ref):
    x = x_ref[...]
    q = jnp.dot(x, wqt_ref[...], preferred_element_type=jnp.float32) + bq_ref[...]
    dn = (((1,), (1,)), ((), ()))
    blk = x.shape[0]
    segs = []
    for half in range(2):
        for h in range(HEADS):
            qh = q[:, h * KD + half * HALF: h * KD + (half + 1) * HALF]
            kh = keys_ref[(2 * h + half) * NK: (2 * h + half + 1) * NK, :]
            segs.append(lax.dot_general(qh, kh, dn,
                                        preferred_element_type=jnp.float32))
    S = jnp.concatenate(segs, axis=0)
    V, I = _topk16(S)
    v1, i1 = V[:4 * blk], I[:4 * blk]
    v2, i2 = V[4 * blk:], I[4 * blk:]
    nb = [KNN // (a + 1) for a in range(KNN)]
    all_s = jnp.concatenate(
        [v1[:, a:a + 1] + v2[:, :nb[a]] for a in range(KNN)]
        + [jnp.full((4 * blk, 14), NEG, jnp.float32)], axis=1)
    all_i = jnp.concatenate(
        [i1[:, a:a + 1] * NK + i2[:, :nb[a]] for a in range(KNN)]
        + [jnp.zeros((4 * blk, 14), jnp.int32)], axis=1)
    v, idx = _topk16(all_s, payload=all_i)
    w = jnp.exp(v - v[:, :1])
    w = w / jnp.sum(w, axis=1, keepdims=True)
    for h in range(HEADS):
        idx_ref[:, h * KNN:(h + 1) * KNN] = idx[h * blk:(h + 1) * blk]
        w_ref[:, h * KNN:(h + 1) * KNN] = w[h * blk:(h + 1) * blk]


def _tc_topk(x_flat, wqt, bq2, keys):
    n = x_flat.shape[0]
    return pl.pallas_call(
        _tc_body,
        grid=(n // BLK,),
        in_specs=[
            pl.BlockSpec((BLK, 1024), lambda i: (i, 0)),
            pl.BlockSpec((1024, HEADS * KD), lambda i: (0, 0)),
            pl.BlockSpec((1, HEADS * KD), lambda i: (0, 0)),
            pl.BlockSpec((2 * HEADS * NK, HALF), lambda i: (0, 0)),
        ],
        out_specs=[
            pl.BlockSpec((BLK, HK), lambda i: (i, 0)),
            pl.BlockSpec((BLK, HK), lambda i: (i, 0)),
        ],
        out_shape=[
            jax.ShapeDtypeStruct((n, HK), jnp.int32),
            jax.ShapeDtypeStruct((n, HK), jnp.float32),
        ],
    )(x_flat, wqt, bq2, keys)


_NC, _NS = 2, 16
_NW = _NC * _NS
_NVS = VD // 16
NCHUNK = 2


@functools.cache
def _make_sc_bag(nt):
    tpw = nt // _NW

    def body(values_hbm, idx_hbm, w_hbm, out_hbm,
             idx_v, w_v, rows_v, acc_v, sg0, sg1, so0, so1):
        wid = lax.axis_index("s") * _NC + lax.axis_index("c")
        base = wid * tpw
        pltpu.sync_copy(idx_hbm.at[pl.ds(base, tpw)], idx_v)
        pltpu.sync_copy(w_hbm.at[pl.ds(base * HK, tpw * HK)], w_v)
        sg = (sg0, sg1)
        so = (so0, so1)
        pltpu.async_copy(values_hbm.at[idx_v.at[0]], rows_v.at[0], sg0)
        pltpu.async_copy(values_hbm.at[idx_v.at[1]], rows_v.at[1], sg1)

        def tbody(i, carry):
            for b in (0, 1):
                t = 2 * i + b
                pltpu.make_async_copy(values_hbm.at[idx_v.at[t]], rows_v.at[b],
                                      sg[b]).wait()

                def jbody(j, acc):
                    wb = plsc.load_gather(
                        w_v, [jnp.full((16,), t * HK + j, dtype=jnp.int32)])
                    return tuple(acc[v] + rows_v[b, j, pl.ds(v * 16, 16)] * wb
                                 for v in range(_NVS))

                acc = lax.fori_loop(
                    0, HK, jbody,
                    tuple(jnp.zeros((16,), jnp.float32) for _ in range(_NVS)))

                @pl.when(t >= 2)
                def _():
                    pltpu.make_async_copy(acc_v.at[b], out_hbm.at[base + t - 2],
                                          so[b]).wait()

                for v in range(_NVS):
                    acc_v[b, pl.ds(v * 16, 16)] = acc[v]
                pltpu.async_copy(acc_v.at[b], out_hbm.at[base + t], so[b])

                @pl.when(t + 2 < tpw)
                def _():
                    pltpu.async_copy(values_hbm.at[idx_v.at[t + 2]], rows_v.at[b],
                                     sg[b])
            return carry

        lax.fori_loop(0, tpw // 2, tbody, 0)
        pltpu.make_async_copy(acc_v.at[0], out_hbm.at[base + tpw - 2], so0).wait()
        pltpu.make_async_copy(acc_v.at[1], out_hbm.at[base + tpw - 1], so1).wait()

    return functools.partial(
        pl.kernel,
        mesh=plsc.VectorSubcoreMesh(core_axis_name="c", subcore_axis_name="s"),
        compiler_params=pltpu.CompilerParams(needs_layout_passes=False),
        out_type=jax.ShapeDtypeStruct((nt, VD), jnp.float32),
        scratch_types=[
            pltpu.VMEM((tpw, HK), jnp.int32),
            pltpu.VMEM((tpw * HK,), jnp.float32),
            pltpu.VMEM((2, HK, VD), jnp.float32),
            pltpu.VMEM((2, VD), jnp.float32),
            pltpu.SemaphoreType.DMA,
            pltpu.SemaphoreType.DMA,
            pltpu.SemaphoreType.DMA,
            pltpu.SemaphoreType.DMA,
        ],
    )(body)


def kernel(x, Wq, bq, keys, values):
    Bb, Tt, C = x.shape
    x_flat = x.reshape(-1, C)
    wqt = Wq.T
    bq2 = bq.reshape(1, -1)
    cbs = BS // NCHUNK
    outs = []
    for c in range(NCHUNK):
        xc = x_flat[c * cbs:(c + 1) * cbs]
        idx, w = _tc_topk(xc, wqt, bq2, keys)
        outs.append(_make_sc_bag(cbs)(values, idx, w.reshape(-1)))
    out = jnp.concatenate(outs, axis=0)
    return out.reshape(Bb, Tt, VD)


# speedup vs baseline: 1.0041x; 1.0041x over previous
import functools

import jax
import jax.numpy as jnp
from jax import lax
from jax.experimental import pallas as pl
from jax.experimental.pallas import tpu as pltpu
from jax.experimental.pallas import tpu_sc as plsc

HEADS = 4
KNN = 16
NK = 512
HALF = 256
KD = 512
BS = 4096
VD = 512
HK = HEADS * KNN
BLK = 128
NEG = -1e30


def _mono_i32(x):
    b = lax.bitcast_convert_type(x, jnp.int32)
    return b ^ (lax.shift_right_arithmetic(b, 31) & jnp.int32(0x7FFFFFFF))


def _inv_mono(k):
    b = k ^ (lax.shift_right_arithmetic(k, 31) & jnp.int32(0x7FFFFFFF))
    return lax.bitcast_convert_type(b, jnp.float32)


def _topk16(s, payload=None):
    n = s.shape[1]
    iota = lax.broadcasted_iota(jnp.int32, s.shape, 1)
    pay = iota if payload is None else payload
    key = _mono_i32(s)
    ms, fps = [], []
    for _ in range(KNN):
        m = jnp.max(key, axis=1, keepdims=True)
        eq = key == m
        fps.append(jnp.min(jnp.where(eq, pay, jnp.int32(2**31 - 1)),
                           axis=1, keepdims=True))
        ms.append(m)
        key = jnp.where(eq, jnp.int32(-2**31), key)
    vals = _inv_mono(jnp.concatenate(ms, axis=1))
    return vals, jnp.concatenate(fps, axis=1)


def _tc_body(x_ref, wqt_ref, bq_ref, keys_ref, idx_ref, w_ref):
    x = x_ref[...]
    q = jnp.dot(x, wqt_ref[...], preferred_element_type=jnp.float32) + bq_ref[...]
    dn = (((1,), (1,)), ((), ()))
    blk = x.shape[0]
    segs = []
    for half in range(2):
        for h in range(HEADS):
            qh = q[:, h * KD + half * HALF: h * KD + (half + 1) * HALF]
            kh = keys_ref[(2 * h + half) * NK: (2 * h + half + 1) * NK, :]
            segs.append(lax.dot_general(qh, kh, dn,
                                        preferred_element_type=jnp.float32))
    S = jnp.concatenate(segs, axis=0)
    V, I = _topk16(S)
    v1, i1 = V[:4 * blk], I[:4 * blk]
    v2, i2 = V[4 * blk:], I[4 * blk:]
    nb = [KNN // (a + 1) for a in range(KNN)]
    all_s = jnp.concatenate(
        [v1[:, a:a + 1] + v2[:, :nb[a]] for a in range(KNN)]
        + [jnp.full((4 * blk, 14), NEG, jnp.float32)], axis=1)
    all_i = jnp.concatenate(
        [i1[:, a:a + 1] * NK + i2[:, :nb[a]] for a in range(KNN)]
        + [jnp.zeros((4 * blk, 14), jnp.int32)], axis=1)
    v, idx = _topk16(all_s, payload=all_i)
    w = jnp.exp(v - v[:, :1])
    w = w / jnp.sum(w, axis=1, keepdims=True)
    for h in range(HEADS):
        idx_ref[:, h * KNN:(h + 1) * KNN] = idx[h * blk:(h + 1) * blk]
        w_ref[:, h * KNN:(h + 1) * KNN] = w[h * blk:(h + 1) * blk]


def _tc_topk(x_flat, wqt, bq2, keys):
    n = x_flat.shape[0]
    return pl.pallas_call(
        _tc_body,
        grid=(n // BLK,),
        in_specs=[
            pl.BlockSpec((BLK, 1024), lambda i: (i, 0)),
            pl.BlockSpec((1024, HEADS * KD), lambda i: (0, 0)),
            pl.BlockSpec((1, HEADS * KD), lambda i: (0, 0)),
            pl.BlockSpec((2 * HEADS * NK, HALF), lambda i: (0, 0)),
        ],
        out_specs=[
            pl.BlockSpec((BLK, HK), lambda i: (i, 0)),
            pl.BlockSpec((BLK, HK), lambda i: (i, 0)),
        ],
        out_shape=[
            jax.ShapeDtypeStruct((n, HK), jnp.int32),
            jax.ShapeDtypeStruct((n, HK), jnp.float32),
        ],
    )(x_flat, wqt, bq2, keys)


_NC, _NS = 2, 16
_NW = _NC * _NS
_NVS = VD // 16
NCHUNK = 4


@functools.cache
def _make_sc_bag(nt):
    tpw = nt // _NW

    def body(values_hbm, idx_hbm, w_hbm, out_hbm,
             idx_v, w_v, rows_v, acc_v, sg0, sg1, so0, so1):
        wid = lax.axis_index("s") * _NC + lax.axis_index("c")
        base = wid * tpw
        pltpu.sync_copy(idx_hbm.at[pl.ds(base, tpw)], idx_v)
        pltpu.sync_copy(w_hbm.at[pl.ds(base * HK, tpw * HK)], w_v)
        sg = (sg0, sg1)
        so = (so0, so1)
        pltpu.async_copy(values_hbm.at[idx_v.at[0]], rows_v.at[0], sg0)
        pltpu.async_copy(values_hbm.at[idx_v.at[1]], rows_v.at[1], sg1)

        def tbody(i, carry):
            for b in (0, 1):
                t = 2 * i + b
                pltpu.make_async_copy(values_hbm.at[idx_v.at[t]], rows_v.at[b],
                                      sg[b]).wait()

                def jbody(j, acc):
                    wb = plsc.load_gather(
                        w_v, [jnp.full((16,), t * HK + j, dtype=jnp.int32)])
                    return tuple(acc[v] + rows_v[b, j, pl.ds(v * 16, 16)] * wb
                                 for v in range(_NVS))

                acc = lax.fori_loop(
                    0, HK, jbody,
                    tuple(jnp.zeros((16,), jnp.float32) for _ in range(_NVS)))

                @pl.when(t >= 2)
                def _():
                    pltpu.make_async_copy(acc_v.at[b], out_hbm.at[base + t - 2],
                                          so[b]).wait()

                for v in range(_NVS):
                    acc_v[b, pl.ds(v * 16, 16)] = acc[v]
                pltpu.async_copy(acc_v.at[b], out_hbm.at[base + t], so[b])

                @pl.when(t + 2 < tpw)
                def _():
                    pltpu.async_copy(values_hbm.at[idx_v.at[t + 2]], rows_v.at[b],
                                     sg[b])
            return carry

        lax.fori_loop(0, tpw // 2, tbody, 0)
        pltpu.make_async_copy(acc_v.at[0], out_hbm.at[base + tpw - 2], so0).wait()
        pltpu.make_async_copy(acc_v.at[1], out_hbm.at[base + tpw - 1], so1).wait()

    return functools.partial(
        pl.kernel,
        mesh=plsc.VectorSubcoreMesh(core_axis_name="c", subcore_axis_name="s"),
        compiler_params=pltpu.CompilerParams(needs_layout_passes=False),
        out_type=jax.ShapeDtypeStruct((nt, VD), jnp.float32),
        scratch_types=[
            pltpu.VMEM((tpw, HK), jnp.int32),
            pltpu.VMEM((tpw * HK,), jnp.float32),
            pltpu.VMEM((2, HK, VD), jnp.float32),
            pltpu.VMEM((2, VD), jnp.float32),
            pltpu.SemaphoreType.DMA,
            pltpu.SemaphoreType.DMA,
            pltpu.SemaphoreType.DMA,
            pltpu.SemaphoreType.DMA,
        ],
    )(body)


def kernel(x, Wq, bq, keys, values):
    Bb, Tt, C = x.shape
    x_flat = x.reshape(-1, C)
    wqt = Wq.T
    bq2 = bq.reshape(1, -1)
    cbs = BS // NCHUNK
    outs = []
    for c in range(NCHUNK):
        xc = x_flat[c * cbs:(c + 1) * cbs]
        idx, w = _tc_topk(xc, wqt, bq2, keys)
        outs.append(_make_sc_bag(cbs)(values, idx, w.reshape(-1)))
    out = jnp.concatenate(outs, axis=0)
    return out.reshape(Bb, Tt, VD)
